# baseline (speedup 1.0000x reference)
import functools

import jax
import jax.numpy as jnp
from jax import lax
from jax.experimental import pallas as pl
from jax.experimental.pallas import tpu as pltpu
from jax.experimental.pallas import tpu_sc as plsc

VOCAB = 1000000
EMBED_DIM = 64
SEQ_LEN = 200
BATCH = 4096

_V2 = VOCAB // 2
_RB = 4096
_GRID = -(-_V2 // _RB)
_TPB = _RB // 128
_RM_LEN = _GRID * _TPB * 256


def _remap_body(x_ref, o_ref):
    v = x_ref[...]
    o_ref[...] = (v & jnp.int32(-256)) | ((v & 1) << 7) | ((v >> 1) & 127)


def _remap(x):
    return pl.pallas_call(
        _remap_body,
        in_specs=[pl.BlockSpec((BATCH, SEQ_LEN), lambda: (0, 0))],
        out_specs=pl.BlockSpec((BATCH, SEQ_LEN), lambda: (0, 0)),
        out_shape=jax.ShapeDtypeStruct((BATCH, SEQ_LEN), jnp.int32),
    )(x)


def _rowmean_body(tab_ref, out_ref):
    a = tab_ref[...]
    lane = lax.broadcasted_iota(jnp.int32, (128, 128), 0)
    col = lax.broadcasted_iota(jnp.int32, (128, 128), 1)
    par = jnp.where((lane // 64) == (col % 2), 1.0 / EMBED_DIM, 0.0)
    z = lax.dot_general(a, par.astype(jnp.float32), (((1,), (0,)), ((), ())),
                        precision=lax.Precision.HIGHEST,
                        preferred_element_type=jnp.float32)
    z3 = z.reshape(_TPB, 128, 128)
    t = jnp.swapaxes(z3, 1, 2)
    out_ref[...] = t[:, 0:2, :]


def _rowmean(table2):
    return pl.pallas_call(
        _rowmean_body,
        grid=(_GRID,),
        in_specs=[pl.BlockSpec((_RB, 128), lambda i: (i, 0))],
        out_specs=pl.BlockSpec((_TPB, 2, 128), lambda i: (i, 0, 0)),
        out_shape=jax.ShapeDtypeStruct((_GRID * _TPB, 2, 128), jnp.float32),
    )(table2)


_NC = 2
_NS = 16
_NW = _NC * _NS
_N_IDX = BATCH * SEQ_LEN
_CHUNK = _N_IDX // _NW


def _gather_body(idx_hbm, rm_hbm, out_hbm, idx_v, val_v, sem):
    wid = lax.axis_index("s") * _NC + lax.axis_index("c")
    base = wid * _CHUNK
    pltpu.sync_copy(idx_hbm.at[pl.ds(base, _CHUNK)], idx_v)
    pltpu.async_copy(rm_hbm.at[idx_v], val_v, sem).wait()
    pltpu.sync_copy(val_v, out_hbm.at[pl.ds(base, _CHUNK)])


def _gather(idx_flat, rowmean):
    mesh = plsc.VectorSubcoreMesh(core_axis_name="c", subcore_axis_name="s")
    f = functools.partial(
        pl.kernel,
        mesh=mesh,
        out_type=jax.ShapeDtypeStruct((_N_IDX,), jnp.float32),
        scratch_types=[
            pltpu.VMEM((_CHUNK,), jnp.int32),
            pltpu.VMEM((_CHUNK,), jnp.float32),
            pltpu.SemaphoreType.DMA,
        ],
    )(_gather_body)
    return f(idx_flat, rowmean)


def _head_body(m_ref, w_ref, b_ref, out_ref):
    z = jnp.sum(m_ref[...] * w_ref[...], axis=1) + b_ref[0]
    out_ref[...] = 1.0 / (1.0 + jnp.exp(-z))


def _head(m, W, b):
    return pl.pallas_call(
        _head_body,
        in_specs=[
            pl.BlockSpec((BATCH, SEQ_LEN), lambda: (0, 0)),
            pl.BlockSpec((1, SEQ_LEN), lambda: (0, 0)),
            pl.BlockSpec(memory_space=pltpu.SMEM),
        ],
        out_specs=pl.BlockSpec((BATCH,), lambda: (0,)),
        out_shape=jax.ShapeDtypeStruct((BATCH,), jnp.float32),
    )(m, W, b)


def kernel(x, table, W, b):
    xp = _remap(x)
    rowmean = _rowmean(table.reshape(_V2, 128)).reshape(-1)
    m = _gather(xp.reshape(-1), rowmean)
    return _head(m.reshape(BATCH, SEQ_LEN), W, b)


# speedup vs baseline: 1.0229x; 1.0229x over previous
import functools

import jax
import jax.numpy as jnp
from jax import lax
from jax.experimental import pallas as pl
from jax.experimental.pallas import tpu as pltpu
from jax.experimental.pallas import tpu_sc as plsc

VOCAB = 1000000
EMBED_DIM = 64
SEQ_LEN = 200
BATCH = 4096

_V2 = VOCAB // 2
_RB = 16384
_GRID = -(-_V2 // _RB)
_TPB = _RB // 128
_RM_LEN = _GRID * _TPB * 256


def _remap_body(x_ref, o_ref):
    v = x_ref[...]
    o_ref[...] = (v & jnp.int32(-256)) | ((v & 1) << 7) | ((v >> 1) & 127)


def _remap(x):
    return pl.pallas_call(
        _remap_body,
        in_specs=[pl.BlockSpec((BATCH, SEQ_LEN), lambda: (0, 0))],
        out_specs=pl.BlockSpec((BATCH, SEQ_LEN), lambda: (0, 0)),
        out_shape=jax.ShapeDtypeStruct((BATCH, SEQ_LEN), jnp.int32),
    )(x)


def _rowmean_body(tab_ref, out_ref):
    a = tab_ref[...]
    lane = lax.broadcasted_iota(jnp.int32, (128, 128), 0)
    col = lax.broadcasted_iota(jnp.int32, (128, 128), 1)
    par = jnp.where((lane // 64) == (col % 2), 1.0 / EMBED_DIM, 0.0)
    z = lax.dot_general(a, par.astype(jnp.float32), (((1,), (0,)), ((), ())),
                        precision=lax.Precision.HIGHEST,
                        preferred_element_type=jnp.float32)
    z3 = z.reshape(_TPB, 128, 128)
    t = jnp.swapaxes(z3, 1, 2)
    out_ref[...] = t[:, 0:2, :]


def _rowmean(table2):
    return pl.pallas_call(
        _rowmean_body,
        grid=(_GRID,),
        in_specs=[pl.BlockSpec((_RB, 128), lambda i: (i, 0))],
        out_specs=pl.BlockSpec((_TPB, 2, 128), lambda i: (i, 0, 0)),
        out_shape=jax.ShapeDtypeStruct((_GRID * _TPB, 2, 128), jnp.float32),
    )(table2)


_NC = 2
_NS = 16
_NW = _NC * _NS
_N_IDX = BATCH * SEQ_LEN
_CHUNK = _N_IDX // _NW


def _gather_body(idx_hbm, rm_hbm, out_hbm, idx_v, val_v, sem):
    wid = lax.axis_index("s") * _NC + lax.axis_index("c")
    base = wid * _CHUNK
    pltpu.sync_copy(idx_hbm.at[pl.ds(base, _CHUNK)], idx_v)
    pltpu.async_copy(rm_hbm.at[idx_v], val_v, sem).wait()
    pltpu.sync_copy(val_v, out_hbm.at[pl.ds(base, _CHUNK)])


def _gather(idx_flat, rowmean):
    mesh = plsc.VectorSubcoreMesh(core_axis_name="c", subcore_axis_name="s")
    f = functools.partial(
        pl.kernel,
        mesh=mesh,
        out_type=jax.ShapeDtypeStruct((_N_IDX,), jnp.float32),
        scratch_types=[
            pltpu.VMEM((_CHUNK,), jnp.int32),
            pltpu.VMEM((_CHUNK,), jnp.float32),
            pltpu.SemaphoreType.DMA,
        ],
    )(_gather_body)
    return f(idx_flat, rowmean)


def _head_body(m_ref, w_ref, b_ref, out_ref):
    z = jnp.sum(m_ref[...] * w_ref[...], axis=1) + b_ref[0]
    out_ref[...] = 1.0 / (1.0 + jnp.exp(-z))


def _head(m, W, b):
    return pl.pallas_call(
        _head_body,
        in_specs=[
            pl.BlockSpec((BATCH, SEQ_LEN), lambda: (0, 0)),
            pl.BlockSpec((1, SEQ_LEN), lambda: (0, 0)),
            pl.BlockSpec(memory_space=pltpu.SMEM),
        ],
        out_specs=pl.BlockSpec((BATCH,), lambda: (0,)),
        out_shape=jax.ShapeDtypeStruct((BATCH,), jnp.float32),
    )(m, W, b)


def kernel(x, table, W, b):
    xp = _remap(x)
    rowmean = _rowmean(table.reshape(_V2, 128)).reshape(-1)
    m = _gather(xp.reshape(-1), rowmean)
    return _head(m.reshape(BATCH, SEQ_LEN), W, b)
